# baseline (speedup 1.0000x reference)
import functools

import jax
import jax.numpy as jnp
from jax import lax
from jax.experimental import pallas as pl
from jax.experimental.pallas import tpu as pltpu
from jax.experimental.pallas import tpu_sc as plsc

NN_NUM = 8
RADIUS_QUERY = 0.08

Q = 1024
KPTS = 100000
BK = 2048
NB = 49
KPAD = NB * BK
NSEL = 10
CAND = 512
INF = 3e38
BIGI = 2**30

NW = 32
QPW = Q // NW
NCH = CAND // 128


def _d2_body(p_ref, ct_ref, d2_ref, ipt_ref, iflat_ref, acc_ref):
    i = pl.program_id(0)
    p = p_ref[...]
    ct = ct_ref[...]
    dotv = lax.dot_general(p, ct, (((1,), (0,)), ((), ())),
                           preferred_element_type=jnp.float32)
    x = p[:, 0:1]
    y = p[:, 1:2]
    z = p[:, 2:3]
    qsq = (x * x + z * z) + y * y
    cx = ct[0:1, :]
    cy = ct[1:2, :]
    cz = ct[2:3, :]
    ksq = (cx * cx + cz * cz) + cy * cy
    d2 = (qsq - 2.0 * dotv) + ksq
    for j in range(BK // 128):
        d2_ref[0, j] = d2[:, 128 * j:128 * (j + 1)]

    @pl.when(i == 0)
    def _():
        acc_ref[...] = d2

    @pl.when(i > 0)
    def _():
        acc_ref[...] = jnp.minimum(acc_ref[...], d2)

    @pl.when(i == NB - 1)
    def _():
        work = acc_ref[...]
        iota = lax.broadcasted_iota(jnp.int32, (Q, BK), 1)
        lanes = []
        for _ in range(NSEL):
            m = jnp.min(work, axis=1, keepdims=True)
            mi = jnp.where(work == m, iota, BIGI)
            lane = jnp.min(mi, axis=1, keepdims=True)
            lanes.append(lane)
            work = jnp.where(iota == lane, INF, work)

        t49 = lax.broadcasted_iota(jnp.int32, (Q, NB), 1)
        cols = [lanes[s] + BK * t49 for s in range(NSEL)]
        cols.append(jnp.zeros((Q, CAND - NSEL * NB), jnp.int32))
        ipt = jnp.concatenate(cols, axis=1)
        ipt_ref[...] = ipt
        qrow = lax.broadcasted_iota(jnp.int32, (Q, CAND), 0) * 128
        iflat_ref[...] = (ipt >> 7) * (Q * 128) + qrow + (ipt & 127)


def _final_body(cand_ref, ipt_ref, d_ref, i_ref, nn_ref):
    work = cand_ref[...]
    ip = ipt_ref[...]
    cpos = lax.broadcasted_iota(jnp.int32, (Q, CAND), 1)
    work = jnp.where(cpos < NSEL * NB, work, INF)
    ds, is_ = [], []
    for _ in range(NN_NUM):
        m = jnp.min(work, axis=1, keepdims=True)
        wi = jnp.where(work == m, ip, BIGI)
        pick = jnp.min(wi, axis=1, keepdims=True)
        ds.append(m)
        is_.append(pick)
        work = jnp.where(wi == pick, INF, work)
    D = jnp.concatenate(ds, axis=1)
    I = jnp.concatenate(is_, axis=1)
    d_ref[...] = D
    i_ref[...] = I
    nn_ref[...] = jnp.sum(
        (D < RADIUS_QUERY ** 2).astype(jnp.int32), axis=1, keepdims=True)


def _make_gather_kernel():
    mesh = plsc.VectorSubcoreMesh(core_axis_name="c", subcore_axis_name="s")

    @functools.partial(
        pl.kernel,
        mesh=mesh,
        out_type=jax.ShapeDtypeStruct((Q, NCH, 128), jnp.float32),
        scratch_types=[
            pltpu.VMEM((2, NCH, 128), jnp.int32),
            pltpu.VMEM((2, NCH, 128), jnp.float32),
            pltpu.SemaphoreType.DMA,
            pltpu.SemaphoreType.DMA,
            pltpu.SemaphoreType.DMA,
            pltpu.SemaphoreType.DMA,
        ],
    )
    def gather_k(d2flat_hbm, idx_hbm, out_hbm, idx_v, val_v,
                 sem_i, sem_g0, sem_g1, sem_o):
        cid = lax.axis_index("c")
        sid = lax.axis_index("s")
        wid = sid * 2 + cid
        q0 = wid * QPW
        sem_g = [sem_g0, sem_g1]

        def load_idx(j, b):
            return pltpu.async_copy(idx_hbm.at[q0 + j], idx_v.at[b], sem_i)

        def fire(b):
            return [
                pltpu.async_copy(
                    d2flat_hbm.at[idx_v.at[b, c]],
                    val_v.at[b, c],
                    sem_g[b],
                )
                for c in range(NCH)
            ]

        load_idx(0, 0).wait()
        cps = fire(0)
        idx_next = load_idx(1, 1)
        outs = [None, None]
        for j in range(QPW):
            b = j % 2
            for cp in cps:
                cp.wait()
            if j + 1 < QPW:
                idx_next.wait()
                if outs[1 - b] is not None:
                    outs[1 - b].wait()
                    outs[1 - b] = None
                cps = fire(1 - b)
                if j + 2 < QPW:
                    idx_next = load_idx(j + 2, b)
            outs[b] = pltpu.async_copy(val_v.at[b], out_hbm.at[q0 + j], sem_o)
        for o in outs:
            if o is not None:
                o.wait()

    return gather_k


def kernel(pos, cloud_pos):
    ct = jnp.concatenate(
        [cloud_pos.T, jnp.full((3, KPAD - KPTS), 1e4, jnp.float32)], axis=1)

    d2, ipt, iflat = pl.pallas_call(
        _d2_body,
        grid=(NB,),
        in_specs=[
            pl.BlockSpec((Q, 3), lambda i: (0, 0)),
            pl.BlockSpec((3, BK), lambda i: (0, i)),
        ],
        out_specs=[
            pl.BlockSpec((1, BK // 128, Q, 128), lambda i: (i, 0, 0, 0)),
            pl.BlockSpec((Q, CAND), lambda i: (0, 0)),
            pl.BlockSpec((Q, CAND), lambda i: (0, 0)),
        ],
        out_shape=[
            jax.ShapeDtypeStruct((NB, BK // 128, Q, 128), jnp.float32),
            jax.ShapeDtypeStruct((Q, CAND), jnp.int32),
            jax.ShapeDtypeStruct((Q, CAND), jnp.int32),
        ],
        scratch_shapes=[pltpu.VMEM((Q, BK), jnp.float32)],
    )(pos, ct)

    gather_k = _make_gather_kernel()
    cand = gather_k(d2.reshape(NB * (BK // 128) * Q * 128),
                    iflat.reshape(Q, NCH, 128))
    cand = cand.reshape(Q, CAND)

    D, I, nn = pl.pallas_call(
        _final_body,
        out_shape=[
            jax.ShapeDtypeStruct((Q, NN_NUM), jnp.float32),
            jax.ShapeDtypeStruct((Q, NN_NUM), jnp.int32),
            jax.ShapeDtypeStruct((Q, 1), jnp.int32),
        ],
    )(cand, ipt)

    return D, I, nn.reshape(Q)


# speedup vs baseline: 1.0448x; 1.0448x over previous
import functools

import jax
import jax.numpy as jnp
from jax import lax
from jax.experimental import pallas as pl
from jax.experimental.pallas import tpu as pltpu
from jax.experimental.pallas import tpu_sc as plsc

NN_NUM = 8
RADIUS_QUERY = 0.08

Q = 1024
KPTS = 100000
BK = 2048
NB = 49
KPAD = NB * BK
NSEL = 10
CAND = 512
INF = 3e38
BIGI = 2**30

NW = 32
QPW = Q // NW
NCH = CAND // 128


def _d2_body(p_ref, ct_ref, d2_ref, ipt_ref, iflat_ref, acc_ref):
    i = pl.program_id(0)
    p = p_ref[...]
    ct = ct_ref[...]
    dotv = lax.dot_general(p, ct, (((1,), (0,)), ((), ())),
                           preferred_element_type=jnp.float32)
    x = p[:, 0:1]
    y = p[:, 1:2]
    z = p[:, 2:3]
    qsq = (x * x + z * z) + y * y
    cx = ct[0:1, :]
    cy = ct[1:2, :]
    cz = ct[2:3, :]
    ksq = (cx * cx + cz * cz) + cy * cy
    d2 = (qsq - 2.0 * dotv) + ksq
    for j in range(BK // 128):
        d2_ref[0, j] = d2[:, 128 * j:128 * (j + 1)]

    @pl.when(i == 0)
    def _():
        acc_ref[...] = d2

    @pl.when(i > 0)
    def _():
        acc_ref[...] = jnp.minimum(acc_ref[...], d2)

    @pl.when(i == NB - 1)
    def _():
        work = acc_ref[...]
        iota = lax.broadcasted_iota(jnp.int32, (Q, BK), 1)
        lanes = []
        for _ in range(NSEL):
            m = jnp.min(work, axis=1, keepdims=True)
            mi = jnp.where(work == m, iota, BIGI)
            lane = jnp.min(mi, axis=1, keepdims=True)
            lanes.append(lane)
            work = jnp.where(iota == lane, INF, work)

        t49 = lax.broadcasted_iota(jnp.int32, (Q, NB), 1)
        cols = [lanes[s] + BK * t49 for s in range(NSEL)]
        cols.append(jnp.zeros((Q, CAND - NSEL * NB), jnp.int32))
        ipt = jnp.concatenate(cols, axis=1)
        ipt_ref[...] = ipt
        qrow = lax.broadcasted_iota(jnp.int32, (Q, CAND), 0) * 128
        iflat_ref[...] = (ipt >> 7) * (Q * 128) + qrow + (ipt & 127)


def _final_body(cand_ref, ipt_ref, d_ref, i_ref, nn_ref):
    work = cand_ref[...]
    ip = ipt_ref[...]
    cpos = lax.broadcasted_iota(jnp.int32, (Q, CAND), 1)
    work = jnp.where(cpos < NSEL * NB, work, INF)
    ds, is_ = [], []
    for _ in range(NN_NUM):
        m = jnp.min(work, axis=1, keepdims=True)
        wi = jnp.where(work == m, ip, BIGI)
        pick = jnp.min(wi, axis=1, keepdims=True)
        ds.append(m)
        is_.append(pick)
        work = jnp.where(wi == pick, INF, work)
    D = jnp.concatenate(ds, axis=1)
    I = jnp.concatenate(is_, axis=1)
    d_ref[...] = D
    i_ref[...] = I
    nn_ref[...] = jnp.sum(
        (D < RADIUS_QUERY ** 2).astype(jnp.int32), axis=1, keepdims=True)


def _make_gather_kernel():
    mesh = plsc.VectorSubcoreMesh(core_axis_name="c", subcore_axis_name="s")

    @functools.partial(
        pl.kernel,
        mesh=mesh,
        out_type=jax.ShapeDtypeStruct((Q, NCH, 128), jnp.float32),
        scratch_types=[
            pltpu.VMEM((2, NCH, 128), jnp.int32),
            pltpu.VMEM((2, NCH, 128), jnp.float32),
            pltpu.SemaphoreType.DMA,
            pltpu.SemaphoreType.DMA,
        ],
    )
    def gather_k(d2flat_hbm, idx_hbm, out_hbm, idx_v, val_v, sem0, sem1):
        cid = lax.axis_index("c")
        sid = lax.axis_index("s")
        wid = sid * 2 + cid
        q0 = wid * QPW
        sems = [sem0, sem1]

        def fire(j, b):
            q = q0 + j
            pltpu.sync_copy(idx_hbm.at[q], idx_v.at[b])
            return [
                pltpu.async_copy(
                    d2flat_hbm.at[idx_v.at[b, c]],
                    val_v.at[b, c],
                    sems[b],
                )
                for c in range(NCH)
            ]

        cps = fire(0, 0)
        for j in range(QPW):
            b = j % 2
            if j + 1 < QPW:
                nxt = fire(j + 1, 1 - b)
            for cp in cps:
                cp.wait()
            pltpu.sync_copy(val_v.at[b], out_hbm.at[q0 + j])
            if j + 1 < QPW:
                cps = nxt

    return gather_k


def kernel(pos, cloud_pos):
    ct = jnp.concatenate(
        [cloud_pos.T, jnp.full((3, KPAD - KPTS), 1e4, jnp.float32)], axis=1)

    d2, ipt, iflat = pl.pallas_call(
        _d2_body,
        grid=(NB,),
        in_specs=[
            pl.BlockSpec((Q, 3), lambda i: (0, 0)),
            pl.BlockSpec((3, BK), lambda i: (0, i)),
        ],
        out_specs=[
            pl.BlockSpec((1, BK // 128, Q, 128), lambda i: (i, 0, 0, 0)),
            pl.BlockSpec((Q, CAND), lambda i: (0, 0)),
            pl.BlockSpec((Q, CAND), lambda i: (0, 0)),
        ],
        out_shape=[
            jax.ShapeDtypeStruct((NB, BK // 128, Q, 128), jnp.float32),
            jax.ShapeDtypeStruct((Q, CAND), jnp.int32),
            jax.ShapeDtypeStruct((Q, CAND), jnp.int32),
        ],
        scratch_shapes=[pltpu.VMEM((Q, BK), jnp.float32)],
    )(pos, ct)

    gather_k = _make_gather_kernel()
    cand = gather_k(d2.reshape(NB * (BK // 128) * Q * 128),
                    iflat.reshape(Q, NCH, 128))
    cand = cand.reshape(Q, CAND)

    D, I, nn = pl.pallas_call(
        _final_body,
        out_shape=[
            jax.ShapeDtypeStruct((Q, NN_NUM), jnp.float32),
            jax.ShapeDtypeStruct((Q, NN_NUM), jnp.int32),
            jax.ShapeDtypeStruct((Q, 1), jnp.int32),
        ],
    )(cand, ipt)

    return D, I, nn.reshape(Q)


# speedup vs baseline: 1.0925x; 1.0456x over previous
import functools

import jax
import jax.numpy as jnp
from jax import lax
from jax.experimental import pallas as pl
from jax.experimental.pallas import tpu as pltpu
from jax.experimental.pallas import tpu_sc as plsc

NN_NUM = 8
RADIUS_QUERY = 0.08

Q = 1024
KPTS = 100000
BK = 2048
NB = 49
KPAD = NB * BK
NSEL = 10
CAND = 512
INF = 3e38
BIGI = 2**30

NW = 32
QPW = Q // NW
NCH = CAND // 128


def _d2_body(p_ref, ct_ref, d2_ref, ipt_ref, iflat_ref, acc_ref):
    i = pl.program_id(0)
    p = p_ref[...]
    ct = ct_ref[...]
    dotv = lax.dot_general(p, ct, (((1,), (0,)), ((), ())),
                           preferred_element_type=jnp.float32)
    x = p[:, 0:1]
    y = p[:, 1:2]
    z = p[:, 2:3]
    qsq = (x * x + z * z) + y * y
    cx = ct[0:1, :]
    cy = ct[1:2, :]
    cz = ct[2:3, :]
    ksq = (cx * cx + cz * cz) + cy * cy
    d2 = (qsq - 2.0 * dotv) + ksq
    for j in range(BK // 128):
        d2_ref[0, j] = d2[:, 128 * j:128 * (j + 1)]

    @pl.when(i == 0)
    def _():
        acc_ref[...] = d2

    @pl.when(i > 0)
    def _():
        acc_ref[...] = jnp.minimum(acc_ref[...], d2)

    @pl.when(i == NB - 1)
    def _():
        work = acc_ref[...]
        iota = lax.broadcasted_iota(jnp.int32, (Q, BK), 1)
        lanes = []
        for _ in range(NSEL):
            m = jnp.min(work, axis=1, keepdims=True)
            mi = jnp.where(work == m, iota, BIGI)
            lane = jnp.min(mi, axis=1, keepdims=True)
            lanes.append(lane)
            work = jnp.where(iota == lane, INF, work)

        t49 = lax.broadcasted_iota(jnp.int32, (Q, NB), 1)
        cols = [lanes[s] + BK * t49 for s in range(NSEL)]
        cols.append(jnp.zeros((Q, CAND - NSEL * NB), jnp.int32))
        ipt = jnp.concatenate(cols, axis=1)
        ipt_ref[...] = ipt
        qrow = lax.broadcasted_iota(jnp.int32, (Q, CAND), 0) * 128
        iflat_ref[...] = (ipt >> 7) * (Q * 128) + qrow + (ipt & 127)


def _final_body(cand_ref, ipt_ref, d_ref, i_ref, nn_ref):
    work = cand_ref[...]
    ip = ipt_ref[...]
    cpos = lax.broadcasted_iota(jnp.int32, (Q, CAND), 1)
    work = jnp.where(cpos < NSEL * NB, work, INF)
    ds, is_ = [], []
    for _ in range(NN_NUM):
        m = jnp.min(work, axis=1, keepdims=True)
        wi = jnp.where(work == m, ip, BIGI)
        pick = jnp.min(wi, axis=1, keepdims=True)
        ds.append(m)
        is_.append(pick)
        work = jnp.where(wi == pick, INF, work)
    D = jnp.concatenate(ds, axis=1)
    I = jnp.concatenate(is_, axis=1)
    d_ref[...] = D
    i_ref[...] = I
    nn_ref[...] = jnp.sum(
        (D < RADIUS_QUERY ** 2).astype(jnp.int32), axis=1, keepdims=True)


def _make_gather_kernel():
    mesh = plsc.VectorSubcoreMesh(core_axis_name="c", subcore_axis_name="s")

    @functools.partial(
        pl.kernel,
        mesh=mesh,
        out_type=jax.ShapeDtypeStruct((Q, NCH, 128), jnp.float32),
        scratch_types=[
            pltpu.VMEM((QPW, NCH, 128), jnp.int32),
            pltpu.VMEM((QPW, NCH, 128), jnp.float32),
            pltpu.SemaphoreType.DMA,
            pltpu.SemaphoreType.DMA,
            pltpu.SemaphoreType.DMA,
        ],
    )
    def gather_k(d2flat_hbm, idx_hbm, out_hbm, idx_all, val_all,
                 sem0, sem1, sem2):
        cid = lax.axis_index("c")
        sid = lax.axis_index("s")
        wid = sid * 2 + cid
        q0 = wid * QPW
        sems = [sem0, sem1, sem2]

        pltpu.sync_copy(idx_hbm.at[pl.ds(q0, QPW)], idx_all)

        def fire(j):
            return [
                pltpu.async_copy(
                    d2flat_hbm.at[idx_all.at[j, c]],
                    val_all.at[j, c],
                    sems[j % 3],
                )
                for c in range(NCH)
            ]

        cps = {0: fire(0), 1: fire(1), 2: fire(2)}
        for j in range(QPW):
            for cp in cps.pop(j):
                cp.wait()
            if j + 3 < QPW:
                cps[j + 3] = fire(j + 3)
        pltpu.sync_copy(val_all, out_hbm.at[pl.ds(q0, QPW)])

    return gather_k


def kernel(pos, cloud_pos):
    ct = jnp.concatenate(
        [cloud_pos.T, jnp.full((3, KPAD - KPTS), 1e4, jnp.float32)], axis=1)

    d2, ipt, iflat = pl.pallas_call(
        _d2_body,
        grid=(NB,),
        in_specs=[
            pl.BlockSpec((Q, 3), lambda i: (0, 0)),
            pl.BlockSpec((3, BK), lambda i: (0, i)),
        ],
        out_specs=[
            pl.BlockSpec((1, BK // 128, Q, 128), lambda i: (i, 0, 0, 0)),
            pl.BlockSpec((Q, CAND), lambda i: (0, 0)),
            pl.BlockSpec((Q, CAND), lambda i: (0, 0)),
        ],
        out_shape=[
            jax.ShapeDtypeStruct((NB, BK // 128, Q, 128), jnp.float32),
            jax.ShapeDtypeStruct((Q, CAND), jnp.int32),
            jax.ShapeDtypeStruct((Q, CAND), jnp.int32),
        ],
        scratch_shapes=[pltpu.VMEM((Q, BK), jnp.float32)],
    )(pos, ct)

    gather_k = _make_gather_kernel()
    cand = gather_k(d2.reshape(NB * (BK // 128) * Q * 128),
                    iflat.reshape(Q, NCH, 128))
    cand = cand.reshape(Q, CAND)

    D, I, nn = pl.pallas_call(
        _final_body,
        out_shape=[
            jax.ShapeDtypeStruct((Q, NN_NUM), jnp.float32),
            jax.ShapeDtypeStruct((Q, NN_NUM), jnp.int32),
            jax.ShapeDtypeStruct((Q, 1), jnp.int32),
        ],
    )(cand, ipt)

    return D, I, nn.reshape(Q)


# speedup vs baseline: 1.1003x; 1.0072x over previous
import functools

import jax
import jax.numpy as jnp
from jax import lax
from jax.experimental import pallas as pl
from jax.experimental.pallas import tpu as pltpu
from jax.experimental.pallas import tpu_sc as plsc

NN_NUM = 8
RADIUS_QUERY = 0.08

Q = 1024
KPTS = 100000
BK = 2048
NB = 49
KPAD = NB * BK
NSEL = 10
CAND = 512
INF = 3e38
BIGI = 2**30

NW = 32
QPW = Q // NW
NCH = CAND // 128


def _d2_body(p_ref, ct_ref, d2_ref, ipt_ref, iflat_ref, acc_ref):
    i = pl.program_id(0)
    p = p_ref[...]
    ct = ct_ref[...]
    dotv = lax.dot_general(p, ct, (((1,), (0,)), ((), ())),
                           preferred_element_type=jnp.float32)
    x = p[:, 0:1]
    y = p[:, 1:2]
    z = p[:, 2:3]
    qsq = (x * x + z * z) + y * y
    cx = ct[0:1, :]
    cy = ct[1:2, :]
    cz = ct[2:3, :]
    ksq = (cx * cx + cz * cz) + cy * cy
    d2 = (qsq - 2.0 * dotv) + ksq
    for j in range(BK // 128):
        d2_ref[0, j] = d2[:, 128 * j:128 * (j + 1)]

    @pl.when(i == 0)
    def _():
        acc_ref[...] = d2

    @pl.when(i > 0)
    def _():
        acc_ref[...] = jnp.minimum(acc_ref[...], d2)

    @pl.when(i == NB - 1)
    def _():
        work = acc_ref[...]
        iota = lax.broadcasted_iota(jnp.int32, (Q, BK), 1)
        lanes = []
        for _ in range(NSEL):
            m = jnp.min(work, axis=1, keepdims=True)
            mi = jnp.where(work == m, iota, BIGI)
            lane = jnp.min(mi, axis=1, keepdims=True)
            lanes.append(lane)
            work = jnp.where(iota == lane, INF, work)

        t49 = lax.broadcasted_iota(jnp.int32, (Q, NB), 1)
        cols = [lanes[s] + BK * t49 for s in range(NSEL)]
        cols.append(jnp.zeros((Q, CAND - NSEL * NB), jnp.int32))
        ipt = jnp.concatenate(cols, axis=1)
        ipt_ref[...] = ipt
        qrow = lax.broadcasted_iota(jnp.int32, (Q, CAND), 0) * 128
        iflat_ref[...] = (ipt >> 7) * (Q * 128) + qrow + (ipt & 127)


def _final_body(cand_ref, ipt_ref, d_ref, i_ref, nn_ref):
    work = cand_ref[...]
    ip = ipt_ref[...]
    cpos = lax.broadcasted_iota(jnp.int32, (Q, CAND), 1)
    work = jnp.where(cpos < NSEL * NB, work, INF)
    ds, is_ = [], []
    for _ in range(NN_NUM):
        m = jnp.min(work, axis=1, keepdims=True)
        wi = jnp.where(work == m, ip, BIGI)
        pick = jnp.min(wi, axis=1, keepdims=True)
        ds.append(m)
        is_.append(pick)
        work = jnp.where(wi == pick, INF, work)
    D = jnp.concatenate(ds, axis=1)
    I = jnp.concatenate(is_, axis=1)
    d_ref[...] = D
    i_ref[...] = I
    nn_ref[...] = jnp.sum(
        (D < RADIUS_QUERY ** 2).astype(jnp.int32), axis=1, keepdims=True)


def _make_gather_kernel():
    mesh = plsc.VectorSubcoreMesh(core_axis_name="c", subcore_axis_name="s")

    @functools.partial(
        pl.kernel,
        mesh=mesh,
        out_type=jax.ShapeDtypeStruct((Q, NCH, 128), jnp.float32),
        scratch_types=[
            pltpu.VMEM((QPW, NCH, 128), jnp.int32),
            pltpu.VMEM((QPW, NCH, 128), jnp.float32),
            pltpu.SemaphoreType.DMA,
            pltpu.SemaphoreType.DMA,
            pltpu.SemaphoreType.DMA,
            pltpu.SemaphoreType.DMA,
            pltpu.SemaphoreType.DMA,
        ],
    )
    def gather_k(d2flat_hbm, idx_hbm, out_hbm, idx_all, val_all,
                 sem0, sem1, sem2, sem3, sem4):
        cid = lax.axis_index("c")
        sid = lax.axis_index("s")
        wid = sid * 2 + cid
        q0 = wid * QPW
        sems = [sem0, sem1, sem2, sem3, sem4]

        pltpu.sync_copy(idx_hbm.at[pl.ds(q0, QPW)], idx_all)

        def fire(j):
            return [
                pltpu.async_copy(
                    d2flat_hbm.at[idx_all.at[j, c]],
                    val_all.at[j, c],
                    sems[j % 5],
                )
                for c in range(NCH)
            ]

        cps = {j: fire(j) for j in range(5)}
        for j in range(QPW):
            for cp in cps.pop(j):
                cp.wait()
            if j + 5 < QPW:
                cps[j + 5] = fire(j + 5)
        pltpu.sync_copy(val_all, out_hbm.at[pl.ds(q0, QPW)])

    return gather_k


def kernel(pos, cloud_pos):
    ct = jnp.concatenate(
        [cloud_pos.T, jnp.full((3, KPAD - KPTS), 1e4, jnp.float32)], axis=1)

    d2, ipt, iflat = pl.pallas_call(
        _d2_body,
        grid=(NB,),
        in_specs=[
            pl.BlockSpec((Q, 3), lambda i: (0, 0)),
            pl.BlockSpec((3, BK), lambda i: (0, i)),
        ],
        out_specs=[
            pl.BlockSpec((1, BK // 128, Q, 128), lambda i: (i, 0, 0, 0)),
            pl.BlockSpec((Q, CAND), lambda i: (0, 0)),
            pl.BlockSpec((Q, CAND), lambda i: (0, 0)),
        ],
        out_shape=[
            jax.ShapeDtypeStruct((NB, BK // 128, Q, 128), jnp.float32),
            jax.ShapeDtypeStruct((Q, CAND), jnp.int32),
            jax.ShapeDtypeStruct((Q, CAND), jnp.int32),
        ],
        scratch_shapes=[pltpu.VMEM((Q, BK), jnp.float32)],
    )(pos, ct)

    gather_k = _make_gather_kernel()
    cand = gather_k(d2.reshape(NB * (BK // 128) * Q * 128),
                    iflat.reshape(Q, NCH, 128))
    cand = cand.reshape(Q, CAND)

    D, I, nn = pl.pallas_call(
        _final_body,
        out_shape=[
            jax.ShapeDtypeStruct((Q, NN_NUM), jnp.float32),
            jax.ShapeDtypeStruct((Q, NN_NUM), jnp.int32),
            jax.ShapeDtypeStruct((Q, 1), jnp.int32),
        ],
    )(cand, ipt)

    return D, I, nn.reshape(Q)
